# baseline (speedup 1.0000x reference)
import functools

import jax
import jax.numpy as jnp
from jax import lax
from jax.experimental import pallas as pl
from jax.experimental.pallas import tpu as pltpu
from jax.experimental.pallas import tpu_sc as plsc

N_NODES = 100000
N_GRAPHS = 1024
D = 128
DJ = D // 16

NC = 2
NS = 16
NW = NC * NS
GPW = N_GRAPHS // NW

B = 160
NBUF = 4
CH = 2000
NCH = N_NODES // CH
NHG = 64
CH_BITS = 11

NEG_INF = -3.4028235e38


def _worker_id():
    return lax.axis_index("s") * NC + lax.axis_index("c")


def _popcnt(mask):
    return plsc.all_reduce_population_count(mask)


def _ffs(mask):
    return plsc.all_reduce_ffs(mask)


def _gather(src_hbm, idx_ref, dst_ref, sem):
    pltpu.async_copy(src_hbm.at[idx_ref], dst_ref, sem).wait()


def _body(x_hbm, batch_hbm, out_hbm, xb0, xb1, xb2, xb3, bb0, bb1, bb2, bb3,
          idxb, hbuf, cbufa, cbufb, stage, semx0, semx1, semx2, semx3):
    xbs = (xb0, xb1, xb2, xb3)
    bbs = (bb0, bb1, bb2, bb3)
    sems = (semx0, semx1, semx2, semx3)
    wid = _worker_id()
    g0 = wid * GPW

    zf = jnp.zeros((16,), jnp.float32)
    lanes = lax.iota(jnp.int32, 16)

    def init_g(g, c):
        for j in range(3 * DJ):
            stage[g, pl.ds(j * 16, 16)] = zf
        return c

    lax.fori_loop(0, GPW, init_g, 0)

    for c in range(NHG // 16):
        gi = jnp.minimum((lanes + 16 * c) * CH, N_NODES - CH)
        idxb[pl.ds(16 * c, 16)] = gi
    _gather(batch_hbm, idxb, hbuf, semx0)

    big = jnp.full((16,), 1 << 30, jnp.int32)

    def chunk_of(T):
        tv = jnp.full((16,), T, jnp.int32)
        acc = jnp.zeros((16,), jnp.int32)
        for c in range(NHG // 16):
            hv = hbuf[pl.ds(16 * c, 16)]
            hv = jnp.where((lanes + 16 * c) < NCH, hv, big)
            acc = acc + _popcnt(hv < tv)
        return jnp.maximum(acc[0] - 1, 0)

    ca = chunk_of(g0)
    cb = chunk_of(g0 + GPW)
    pltpu.make_async_copy(batch_hbm.at[pl.ds(ca * CH, CH)],
                          cbufa.at[pl.ds(0, CH)], semx0).start()
    pltpu.make_async_copy(batch_hbm.at[pl.ds(cb * CH, CH)],
                          cbufb.at[pl.ds(0, CH)], semx1).start()
    pltpu.make_async_copy(batch_hbm.at[pl.ds(0, CH)],
                          cbufa.at[pl.ds(0, CH)], semx0).wait()
    pltpu.make_async_copy(batch_hbm.at[pl.ds(0, CH)],
                          cbufb.at[pl.ds(0, CH)], semx1).wait()
    cbufa[pl.ds(CH, 16)] = big
    cbufb[pl.ds(CH, 16)] = big

    def lower_bound(cbuf, base, T):
        def step(i, lh):
            lo_s, hi_s = lh
            mid = (lo_s + hi_s) // 2
            v = cbuf[pl.ds(mid, 16)][0]
            lt = v < T
            lo_s = jnp.where(lt, mid + 1, lo_s)
            hi_s = jnp.where(lt, hi_s, mid)
            return (lo_s, hi_s)

        lo_s, _ = lax.fori_loop(0, CH_BITS, step,
                                (jnp.int32(0), jnp.int32(CH)))
        return base + lo_s

    lo = lower_bound(cbufa, ca * CH, g0)
    hi = lower_bound(cbufb, cb * CH, g0 + GPW)

    blk0 = (lo // B) * B
    nblk = (hi - blk0 + (B - 1)) // B

    def issue(blk, xb, bb, sem):
        pltpu.make_async_copy(x_hbm.at[pl.ds(blk, B)], xb, sem).start()
        pltpu.make_async_copy(batch_hbm.at[pl.ds(blk, B)],
                              bb.at[pl.ds(0, B)], sem).start()

    def drain(xb, bb, sem):
        pltpu.make_async_copy(x_hbm.at[pl.ds(0, B)], xb, sem).wait()
        pltpu.make_async_copy(batch_hbm.at[pl.ds(0, B)],
                              bb.at[pl.ds(0, B)], sem).wait()

    def flush(prev_id, cnt_s, s, m):
        bl = prev_id - g0
        cntf = jnp.full((16,), cnt_s).astype(jnp.float32)
        inv = 1.0 / cntf
        for j in range(DJ):
            stage[bl, pl.ds(j * 16, 16)] = s[j] * inv
            stage[bl, pl.ds(D + j * 16, 16)] = m[j]
            stage[bl, pl.ds(2 * D + j * 16, 16)] = s[j]

    lanes = lax.iota(jnp.int32, 16)

    def process(blk, xb, bb, carry):
        r0 = jnp.minimum(jnp.maximum(lo - blk, 0), B)
        r1 = jnp.minimum(jnp.maximum(hi - blk, 0), B)

        def run_cond(st):
            return st[0] < r1

        def run_body(st):
            r, prev_id, cnt_s, s, m = st
            cur = bb[pl.ds(r, 16)][0]
            changed = cur != prev_id

            @pl.when(changed & (cnt_s > 0))
            def _():
                flush(prev_id, cnt_s, s, m)

            s = tuple(jnp.where(changed, zf, sj) for sj in s)
            m = tuple(jnp.where(changed, NEG_INF, mj) for mj in m)
            cnt_s = jnp.where(changed, 0, cnt_s)

            cur_v = jnp.full((16,), cur, jnp.int32)

            def se_cond(st2):
                return (st2[1] < 0) & (st2[0] < r1)

            def se_body(st2):
                rr, _e = st2
                chunk = bb[pl.ds(rr, 16)]
                mm = (chunk != cur_v) | ((lanes + rr) >= r1)
                pc = _popcnt(mm)
                fi = _ffs(mm)
                e_v = jnp.where(pc > 0, rr + fi, -1)
                return (rr + 16, e_v[0])

            _rr, e = lax.while_loop(se_cond, se_body, (r, jnp.int32(-1)))
            e = jnp.where(e < 0, r1, e)

            def acc_rows(base, nrows, sm):
                s2, m2 = sm
                s3, m3 = list(s2), list(m2)
                for u in range(nrows):
                    for j in range(DJ):
                        v = xb[base + u, pl.ds(j * 16, 16)]
                        s3[j] = s3[j] + v
                        m3[j] = jnp.maximum(m3[j], v)
                return (tuple(s3), tuple(m3))

            n4 = (e - r) // 4

            def acc4(k, sm):
                return acc_rows(r + 4 * k, 4, sm)

            def acc1(rr, sm):
                return acc_rows(rr, 1, sm)

            s, m = lax.fori_loop(0, n4, acc4, (s, m))
            s, m = lax.fori_loop(r + 4 * n4, e, acc1, (s, m))
            cnt_s = cnt_s + (e - r)
            return (e, cur, cnt_s, s, m)

        r, prev_id, cnt_s, s, m = lax.while_loop(
            run_cond, run_body, (r0,) + carry)
        return (prev_id, cnt_s, s, m)

    carry0 = (jnp.int32(-1), jnp.int32(0),
              tuple(zf for _ in range(DJ)),
              tuple(jnp.full((16,), NEG_INF, jnp.float32) for _ in range(DJ)))

    for u in range(NBUF - 1):
        @pl.when(u < nblk)
        def _(u=u):
            issue(blk0 + u * B, xbs[u], bbs[u], sems[u])

    ngrp = (nblk + NBUF - 1) // NBUF

    def do_grp(g, carry):
        for u in range(NBUF):
            b = NBUF * g + u
            blk = blk0 + b * B
            pre = b + (NBUF - 1)

            @pl.when(pre < nblk)
            def _(pre=pre, u=u):
                issue(blk0 + pre * B, xbs[(u + NBUF - 1) % NBUF],
                      bbs[(u + NBUF - 1) % NBUF], sems[(u + NBUF - 1) % NBUF])

            @pl.when(b < nblk)
            def _(u=u):
                drain(xbs[u], bbs[u], sems[u])

            carry = process(blk, xbs[u], bbs[u], carry)
        return carry

    prev_id, cnt_s, s, m = lax.fori_loop(0, ngrp, do_grp, carry0)

    @pl.when(cnt_s > 0)
    def _():
        flush(prev_id, cnt_s, s, m)

    pltpu.sync_copy(stage, out_hbm.at[pl.ds(g0, GPW)])


@jax.jit
def _pool(x, batch):
    mesh = plsc.VectorSubcoreMesh(core_axis_name="c", subcore_axis_name="s",
                                  num_cores=NC, num_subcores=NS)
    run = pl.kernel(
        _body,
        out_type=jax.ShapeDtypeStruct((N_GRAPHS, 3 * D), jnp.float32),
        mesh=mesh,
        compiler_params=pltpu.CompilerParams(needs_layout_passes=False),
        scratch_types=(
            [pltpu.VMEM((B, D), jnp.float32)] * NBUF
            + [pltpu.VMEM((B + 16,), jnp.int32)] * NBUF
            + [
                pltpu.VMEM((NHG,), jnp.int32),
                pltpu.VMEM((NHG,), jnp.int32),
                pltpu.VMEM((CH + 16,), jnp.int32),
                pltpu.VMEM((CH + 16,), jnp.int32),
                pltpu.VMEM((GPW, 3 * D), jnp.float32),
            ]
            + [pltpu.SemaphoreType.DMA] * NBUF
        ),
    )
    return run(x, batch)


def kernel(x, batch):
    return _pool(x, batch.astype(jnp.int32))


# speedup vs baseline: 1.0249x; 1.0249x over previous
import functools

import jax
import jax.numpy as jnp
from jax import lax
from jax.experimental import pallas as pl
from jax.experimental.pallas import tpu as pltpu
from jax.experimental.pallas import tpu_sc as plsc

N_NODES = 100000
N_GRAPHS = 1024
D = 128
DJ = D // 16

NC = 2
NS = 16
NW = NC * NS
GPW = N_GRAPHS // NW

B = 400
NBUF = 2
CH = 2000
NCH = N_NODES // CH
NHG = 64
CH_BITS = 11

NEG_INF = -3.4028235e38


def _worker_id():
    return lax.axis_index("s") * NC + lax.axis_index("c")


def _popcnt(mask):
    return plsc.all_reduce_population_count(mask)


def _ffs(mask):
    return plsc.all_reduce_ffs(mask)


def _gather(src_hbm, idx_ref, dst_ref, sem):
    pltpu.async_copy(src_hbm.at[idx_ref], dst_ref, sem).wait()


def _body(x_hbm, batch_hbm, out_hbm, *scr):
    xbs = scr[0:NBUF]
    bbs = scr[NBUF:2 * NBUF]
    idxb, hbuf, cbufa, cbufb, stage = scr[2 * NBUF:2 * NBUF + 5]
    sems = scr[2 * NBUF + 5:]
    semx0, semx1 = sems[0], sems[1]
    wid = _worker_id()
    g0 = wid * GPW

    zf = jnp.zeros((16,), jnp.float32)
    lanes = lax.iota(jnp.int32, 16)

    def init_g(g, c):
        for j in range(3 * DJ):
            stage[g, pl.ds(j * 16, 16)] = zf
        return c

    lax.fori_loop(0, GPW, init_g, 0)

    for c in range(NHG // 16):
        gi = jnp.minimum((lanes + 16 * c) * CH, N_NODES - CH)
        idxb[pl.ds(16 * c, 16)] = gi
    _gather(batch_hbm, idxb, hbuf, semx0)

    big = jnp.full((16,), 1 << 30, jnp.int32)

    def chunk_of(T):
        tv = jnp.full((16,), T, jnp.int32)
        acc = jnp.zeros((16,), jnp.int32)
        for c in range(NHG // 16):
            hv = hbuf[pl.ds(16 * c, 16)]
            hv = jnp.where((lanes + 16 * c) < NCH, hv, big)
            acc = acc + _popcnt(hv < tv)
        return jnp.maximum(acc[0] - 1, 0)

    ca = chunk_of(g0)
    cb = chunk_of(g0 + GPW)
    pltpu.make_async_copy(batch_hbm.at[pl.ds(ca * CH, CH)],
                          cbufa.at[pl.ds(0, CH)], semx0).start()
    pltpu.make_async_copy(batch_hbm.at[pl.ds(cb * CH, CH)],
                          cbufb.at[pl.ds(0, CH)], semx1).start()
    pltpu.make_async_copy(batch_hbm.at[pl.ds(0, CH)],
                          cbufa.at[pl.ds(0, CH)], semx0).wait()
    pltpu.make_async_copy(batch_hbm.at[pl.ds(0, CH)],
                          cbufb.at[pl.ds(0, CH)], semx1).wait()
    cbufa[pl.ds(CH, 16)] = big
    cbufb[pl.ds(CH, 16)] = big

    def lower_bound(cbuf, base, T):
        def step(i, lh):
            lo_s, hi_s = lh
            mid = (lo_s + hi_s) // 2
            v = cbuf[pl.ds(mid, 16)][0]
            lt = v < T
            lo_s = jnp.where(lt, mid + 1, lo_s)
            hi_s = jnp.where(lt, hi_s, mid)
            return (lo_s, hi_s)

        lo_s, _ = lax.fori_loop(0, CH_BITS, step,
                                (jnp.int32(0), jnp.int32(CH)))
        return base + lo_s

    lo = lower_bound(cbufa, ca * CH, g0)
    hi = lower_bound(cbufb, cb * CH, g0 + GPW)

    blk0 = (lo // B) * B
    nblk = (hi - blk0 + (B - 1)) // B

    def issue(blk, xb, bb, sem):
        pltpu.make_async_copy(x_hbm.at[pl.ds(blk, B)], xb, sem).start()
        pltpu.make_async_copy(batch_hbm.at[pl.ds(blk, B)],
                              bb.at[pl.ds(0, B)], sem).start()

    def drain(xb, bb, sem):
        pltpu.make_async_copy(x_hbm.at[pl.ds(0, B)], xb, sem).wait()
        pltpu.make_async_copy(batch_hbm.at[pl.ds(0, B)],
                              bb.at[pl.ds(0, B)], sem).wait()

    def flush(prev_id, cnt_s, s, m):
        bl = prev_id - g0
        cntf = jnp.full((16,), cnt_s).astype(jnp.float32)
        inv = 1.0 / cntf
        for j in range(DJ):
            stage[bl, pl.ds(j * 16, 16)] = s[j] * inv
            stage[bl, pl.ds(D + j * 16, 16)] = m[j]
            stage[bl, pl.ds(2 * D + j * 16, 16)] = s[j]

    lanes = lax.iota(jnp.int32, 16)

    def process(blk, xb, bb, carry):
        r0 = jnp.minimum(jnp.maximum(lo - blk, 0), B)
        r1 = jnp.minimum(jnp.maximum(hi - blk, 0), B)

        def run_cond(st):
            return st[0] < r1

        def run_body(st):
            r, prev_id, cnt_s, s, m = st
            cur = bb[pl.ds(r, 16)][0]
            changed = cur != prev_id

            @pl.when(changed & (cnt_s > 0))
            def _():
                flush(prev_id, cnt_s, s, m)

            s = tuple(jnp.where(changed, zf, sj) for sj in s)
            m = tuple(jnp.where(changed, NEG_INF, mj) for mj in m)
            cnt_s = jnp.where(changed, 0, cnt_s)

            cur_v = jnp.full((16,), cur, jnp.int32)

            def se_cond(st2):
                return (st2[1] < 0) & (st2[0] < r1)

            def se_body(st2):
                rr, _e = st2
                chunk = bb[pl.ds(rr, 16)]
                mm = (chunk != cur_v) | ((lanes + rr) >= r1)
                pc = _popcnt(mm)
                fi = _ffs(mm)
                e_v = jnp.where(pc > 0, rr + fi, -1)
                return (rr + 16, e_v[0])

            _rr, e = lax.while_loop(se_cond, se_body, (r, jnp.int32(-1)))
            e = jnp.where(e < 0, r1, e)

            def acc_rows(base, nrows, sm):
                s2, m2 = sm
                s3, m3 = list(s2), list(m2)
                for u in range(nrows):
                    for j in range(DJ):
                        v = xb[base + u, pl.ds(j * 16, 16)]
                        s3[j] = s3[j] + v
                        m3[j] = jnp.maximum(m3[j], v)
                return (tuple(s3), tuple(m3))

            n4 = (e - r) // 4

            def acc4(k, sm):
                return acc_rows(r + 4 * k, 4, sm)

            def acc1(rr, sm):
                return acc_rows(rr, 1, sm)

            s, m = lax.fori_loop(0, n4, acc4, (s, m))
            s, m = lax.fori_loop(r + 4 * n4, e, acc1, (s, m))
            cnt_s = cnt_s + (e - r)
            return (e, cur, cnt_s, s, m)

        r, prev_id, cnt_s, s, m = lax.while_loop(
            run_cond, run_body, (r0,) + carry)
        return (prev_id, cnt_s, s, m)

    carry0 = (jnp.int32(-1), jnp.int32(0),
              tuple(zf for _ in range(DJ)),
              tuple(jnp.full((16,), NEG_INF, jnp.float32) for _ in range(DJ)))

    for u in range(NBUF - 1):
        @pl.when(u < nblk)
        def _(u=u):
            issue(blk0 + u * B, xbs[u], bbs[u], sems[u])

    ngrp = (nblk + NBUF - 1) // NBUF

    def do_grp(g, carry):
        for u in range(NBUF):
            b = NBUF * g + u
            blk = blk0 + b * B
            pre = b + (NBUF - 1)

            @pl.when(pre < nblk)
            def _(pre=pre, u=u):
                issue(blk0 + pre * B, xbs[(u + NBUF - 1) % NBUF],
                      bbs[(u + NBUF - 1) % NBUF], sems[(u + NBUF - 1) % NBUF])

            @pl.when(b < nblk)
            def _(u=u):
                drain(xbs[u], bbs[u], sems[u])

            carry = process(blk, xbs[u], bbs[u], carry)
        return carry

    prev_id, cnt_s, s, m = lax.fori_loop(0, ngrp, do_grp, carry0)

    @pl.when(cnt_s > 0)
    def _():
        flush(prev_id, cnt_s, s, m)

    pltpu.sync_copy(stage, out_hbm.at[pl.ds(g0, GPW)])


@jax.jit
def _pool(x, batch):
    mesh = plsc.VectorSubcoreMesh(core_axis_name="c", subcore_axis_name="s",
                                  num_cores=NC, num_subcores=NS)
    run = pl.kernel(
        _body,
        out_type=jax.ShapeDtypeStruct((N_GRAPHS, 3 * D), jnp.float32),
        mesh=mesh,
        compiler_params=pltpu.CompilerParams(needs_layout_passes=False),
        scratch_types=(
            [pltpu.VMEM((B, D), jnp.float32)] * NBUF
            + [pltpu.VMEM((B + 16,), jnp.int32)] * NBUF
            + [
                pltpu.VMEM((NHG,), jnp.int32),
                pltpu.VMEM((NHG,), jnp.int32),
                pltpu.VMEM((CH + 16,), jnp.int32),
                pltpu.VMEM((CH + 16,), jnp.int32),
                pltpu.VMEM((GPW, 3 * D), jnp.float32),
            ]
            + [pltpu.SemaphoreType.DMA] * NBUF
        ),
    )
    return run(x, batch)


def kernel(x, batch):
    return _pool(x, batch.astype(jnp.int32))


# speedup vs baseline: 1.0319x; 1.0068x over previous
import functools

import jax
import jax.numpy as jnp
from jax import lax
from jax.experimental import pallas as pl
from jax.experimental.pallas import tpu as pltpu
from jax.experimental.pallas import tpu_sc as plsc

N_NODES = 100000
N_GRAPHS = 1024
D = 128
DJ = D // 16

NC = 2
NS = 16
NW = NC * NS
GPW = N_GRAPHS // NW

B = 400
NBUF = 2
CH = 2000
NCH = N_NODES // CH
NHG = 64
CH_BITS = 11

NEG_INF = -3.4028235e38


def _worker_id():
    return lax.axis_index("s") * NC + lax.axis_index("c")


def _popcnt(mask):
    return plsc.all_reduce_population_count(mask)


def _ffs(mask):
    return plsc.all_reduce_ffs(mask)


def _gather_start(src_hbm, idx_ref, dst_ref, sem):
    pltpu.make_async_copy(src_hbm.at[idx_ref], dst_ref, sem).start()


def _gather_wait(src_hbm, idx_ref, dst_ref, sem):
    pltpu.make_async_copy(src_hbm.at[idx_ref], dst_ref, sem).wait()


def _body(x_hbm, batch_hbm, out_hbm, *scr):
    xbs = scr[0:NBUF]
    bbs = scr[NBUF:2 * NBUF]
    idxb, hbuf, cbufa, cbufb, stage = scr[2 * NBUF:2 * NBUF + 5]
    sems = scr[2 * NBUF + 5:]
    semx0, semx1 = sems[0], sems[1]
    wid = _worker_id()
    g0 = wid * GPW

    zf = jnp.zeros((16,), jnp.float32)
    lanes = lax.iota(jnp.int32, 16)

    for c in range(NHG // 16):
        gi = jnp.minimum((lanes + 16 * c) * CH, N_NODES - CH)
        idxb[pl.ds(16 * c, 16)] = gi
    _gather_start(batch_hbm, idxb, hbuf, semx0)

    def init_g(g, c):
        for j in range(3 * DJ):
            stage[g, pl.ds(j * 16, 16)] = zf
        return c

    lax.fori_loop(0, GPW, init_g, 0)
    _gather_wait(batch_hbm, idxb, hbuf, semx0)

    big = jnp.full((16,), 1 << 30, jnp.int32)

    def chunk_of(T):
        tv = jnp.full((16,), T, jnp.int32)
        acc = jnp.zeros((16,), jnp.int32)
        for c in range(NHG // 16):
            hv = hbuf[pl.ds(16 * c, 16)]
            hv = jnp.where((lanes + 16 * c) < NCH, hv, big)
            acc = acc + _popcnt(hv < tv)
        return jnp.maximum(acc[0] - 1, 0)

    ca = chunk_of(g0)
    cb = chunk_of(g0 + GPW)
    pltpu.make_async_copy(batch_hbm.at[pl.ds(ca * CH, CH)],
                          cbufa.at[pl.ds(0, CH)], semx0).start()
    pltpu.make_async_copy(batch_hbm.at[pl.ds(cb * CH, CH)],
                          cbufb.at[pl.ds(0, CH)], semx1).start()
    pltpu.make_async_copy(batch_hbm.at[pl.ds(0, CH)],
                          cbufa.at[pl.ds(0, CH)], semx0).wait()
    pltpu.make_async_copy(batch_hbm.at[pl.ds(0, CH)],
                          cbufb.at[pl.ds(0, CH)], semx1).wait()
    cbufa[pl.ds(CH, 16)] = big
    cbufb[pl.ds(CH, 16)] = big

    def lower_bound(cbuf, base, T):
        def step(i, lh):
            lo_s, hi_s = lh
            mid = (lo_s + hi_s) // 2
            v = cbuf[pl.ds(mid, 16)][0]
            lt = v < T
            lo_s = jnp.where(lt, mid + 1, lo_s)
            hi_s = jnp.where(lt, hi_s, mid)
            return (lo_s, hi_s)

        lo_s, _ = lax.fori_loop(0, CH_BITS, step,
                                (jnp.int32(0), jnp.int32(CH)))
        return base + lo_s

    lo = lower_bound(cbufa, ca * CH, g0)
    hi = lower_bound(cbufb, cb * CH, g0 + GPW)

    blk0 = (lo // B) * B
    nblk = (hi - blk0 + (B - 1)) // B

    def issue(blk, xb, bb, sem):
        pltpu.make_async_copy(x_hbm.at[pl.ds(blk, B)], xb, sem).start()
        pltpu.make_async_copy(batch_hbm.at[pl.ds(blk, B)],
                              bb.at[pl.ds(0, B)], sem).start()

    def drain(xb, bb, sem):
        pltpu.make_async_copy(x_hbm.at[pl.ds(0, B)], xb, sem).wait()
        pltpu.make_async_copy(batch_hbm.at[pl.ds(0, B)],
                              bb.at[pl.ds(0, B)], sem).wait()

    def flush(prev_id, cnt_s, s, m):
        bl = prev_id - g0
        cntf = jnp.full((16,), cnt_s).astype(jnp.float32)
        inv = 1.0 / cntf
        for j in range(DJ):
            stage[bl, pl.ds(j * 16, 16)] = s[j] * inv
            stage[bl, pl.ds(D + j * 16, 16)] = m[j]
            stage[bl, pl.ds(2 * D + j * 16, 16)] = s[j]

    lanes = lax.iota(jnp.int32, 16)

    def process(blk, xb, bb, carry):
        r0 = jnp.minimum(jnp.maximum(lo - blk, 0), B)
        r1 = jnp.minimum(jnp.maximum(hi - blk, 0), B)

        def run_cond(st):
            return st[0] < r1

        def run_body(st):
            r, prev_id, cnt_s, s, m = st
            cur = bb[pl.ds(r, 16)][0]
            changed = cur != prev_id

            @pl.when(changed & (cnt_s > 0))
            def _():
                flush(prev_id, cnt_s, s, m)

            s = tuple(jnp.where(changed, zf, sj) for sj in s)
            m = tuple(jnp.where(changed, NEG_INF, mj) for mj in m)
            cnt_s = jnp.where(changed, 0, cnt_s)

            cur_v = jnp.full((16,), cur, jnp.int32)

            def se_cond(st2):
                return (st2[1] < 0) & (st2[0] < r1)

            def se_body(st2):
                rr, _e = st2
                chunk = bb[pl.ds(rr, 16)]
                mm = (chunk != cur_v) | ((lanes + rr) >= r1)
                pc = _popcnt(mm)
                fi = _ffs(mm)
                e_v = jnp.where(pc > 0, rr + fi, -1)
                return (rr + 16, e_v[0])

            _rr, e = lax.while_loop(se_cond, se_body, (r, jnp.int32(-1)))
            e = jnp.where(e < 0, r1, e)

            def acc_rows(base, nrows, sm):
                s2, m2 = sm
                s3, m3 = list(s2), list(m2)
                for u in range(nrows):
                    for j in range(DJ):
                        v = xb[base + u, pl.ds(j * 16, 16)]
                        s3[j] = s3[j] + v
                        m3[j] = jnp.maximum(m3[j], v)
                return (tuple(s3), tuple(m3))

            n4 = (e - r) // 4

            def acc4(k, sm):
                return acc_rows(r + 4 * k, 4, sm)

            def acc1(rr, sm):
                return acc_rows(rr, 1, sm)

            s, m = lax.fori_loop(0, n4, acc4, (s, m))
            s, m = lax.fori_loop(r + 4 * n4, e, acc1, (s, m))
            cnt_s = cnt_s + (e - r)
            return (e, cur, cnt_s, s, m)

        r, prev_id, cnt_s, s, m = lax.while_loop(
            run_cond, run_body, (r0,) + carry)
        return (prev_id, cnt_s, s, m)

    carry0 = (jnp.int32(-1), jnp.int32(0),
              tuple(zf for _ in range(DJ)),
              tuple(jnp.full((16,), NEG_INF, jnp.float32) for _ in range(DJ)))

    for u in range(NBUF - 1):
        @pl.when(u < nblk)
        def _(u=u):
            issue(blk0 + u * B, xbs[u], bbs[u], sems[u])

    ngrp = (nblk + NBUF - 1) // NBUF

    def do_grp(g, carry):
        for u in range(NBUF):
            b = NBUF * g + u
            blk = blk0 + b * B
            pre = b + (NBUF - 1)

            @pl.when(pre < nblk)
            def _(pre=pre, u=u):
                issue(blk0 + pre * B, xbs[(u + NBUF - 1) % NBUF],
                      bbs[(u + NBUF - 1) % NBUF], sems[(u + NBUF - 1) % NBUF])

            @pl.when(b < nblk)
            def _(u=u):
                drain(xbs[u], bbs[u], sems[u])

            carry = process(blk, xbs[u], bbs[u], carry)
        return carry

    prev_id, cnt_s, s, m = lax.fori_loop(0, ngrp, do_grp, carry0)

    @pl.when(cnt_s > 0)
    def _():
        flush(prev_id, cnt_s, s, m)

    pltpu.sync_copy(stage, out_hbm.at[pl.ds(g0, GPW)])


@jax.jit
def _pool(x, batch):
    mesh = plsc.VectorSubcoreMesh(core_axis_name="c", subcore_axis_name="s",
                                  num_cores=NC, num_subcores=NS)
    run = pl.kernel(
        _body,
        out_type=jax.ShapeDtypeStruct((N_GRAPHS, 3 * D), jnp.float32),
        mesh=mesh,
        compiler_params=pltpu.CompilerParams(needs_layout_passes=False),
        scratch_types=(
            [pltpu.VMEM((B, D), jnp.float32)] * NBUF
            + [pltpu.VMEM((B + 16,), jnp.int32)] * NBUF
            + [
                pltpu.VMEM((NHG,), jnp.int32),
                pltpu.VMEM((NHG,), jnp.int32),
                pltpu.VMEM((CH + 16,), jnp.int32),
                pltpu.VMEM((CH + 16,), jnp.int32),
                pltpu.VMEM((GPW, 3 * D), jnp.float32),
            ]
            + [pltpu.SemaphoreType.DMA] * NBUF
        ),
    )
    return run(x, batch)


def kernel(x, batch):
    return _pool(x, batch.astype(jnp.int32))
